# Initial kernel scaffold; baseline (speedup 1.0000x reference)
import jax
import jax.numpy as jnp
from jax.experimental import pallas as pl


def kernel(seq, emb, W, b):
    raise NotImplementedError("write your pallas kernel here")


# speedup vs baseline: 4.9937x; 4.9937x over previous
import functools

import jax
import jax.numpy as jnp
from jax import lax
from jax.experimental import pallas as pl
from jax.experimental.pallas import tpu as pltpu
from jax.experimental.pallas import tpu_sc as plsc

_VOCAB = 100000
_EMB = 128
_B = 4096
_L = 50
_D = 16
_NC = 2
_NS = 16
_NW = _NC * _NS
_SEQ_PER_W = _B // _NW
_CHUNK_SEQ = 8
_CHUNK_TOK = _CHUNK_SEQ * _L
_SUB = 4
_SUB_TOK = _CHUNK_TOK // _SUB
_N_CHUNKS = _SEQ_PER_W // _CHUNK_SEQ
_PROJ_BLK = 4000


def _proj_body(emb_ref, w_ref, out_ref):
    out_ref[...] = jnp.dot(emb_ref[...], w_ref[...],
                           preferred_element_type=jnp.float32)


def _project_table(emb, w_pad):
    return pl.pallas_call(
        _proj_body,
        grid=(_VOCAB // _PROJ_BLK,),
        in_specs=[
            pl.BlockSpec((_PROJ_BLK, _EMB), lambda i: (i, 0)),
            pl.BlockSpec((_EMB, _D), lambda i: (0, 0)),
        ],
        out_specs=pl.BlockSpec((_PROJ_BLK, _D), lambda i: (i, 0)),
        out_shape=jax.ShapeDtypeStruct((_VOCAB, _D), jnp.float32),
    )(emb, w_pad)


@functools.partial(
    pl.kernel,
    out_type=jax.ShapeDtypeStruct((_B, _D), jnp.float32),
    mesh=plsc.VectorSubcoreMesh(core_axis_name="c", subcore_axis_name="s",
                                num_cores=_NC, num_subcores=_NS),
    scratch_types=[
        pltpu.VMEM((_SUB, _SUB_TOK), jnp.int32),
        pltpu.VMEM((_SUB, _SUB_TOK, _D), jnp.float32),
        pltpu.VMEM((_SEQ_PER_W, _D), jnp.float32),
        pltpu.VMEM((_D,), jnp.float32),
        pltpu.SemaphoreType.DMA,
    ],
    compiler_params=pltpu.CompilerParams(use_tc_tiling_on_sc=False),
)
def _sc_pool(seq_hbm, p_hbm, bvec_hbm, out_hbm, idx_v, rows_v, out_v,
             bvec_v, sem):
    wid = lax.axis_index("s") * _NC + lax.axis_index("c")
    pltpu.sync_copy(bvec_hbm, bvec_v)
    bvec = bvec_v[...]

    def chunk_body(c, carry):
        gchunk = wid * _N_CHUNKS + c
        pltpu.sync_copy(seq_hbm.at[gchunk], idx_v)
        for j in range(_SUB):
            pltpu.async_copy(p_hbm.at[idx_v.at[j]], rows_v.at[j], sem).wait()
        for s in range(_CHUNK_SEQ):
            j = s // 2
            half = (s % 2) * _L

            def add_body(t, acc):
                return acc + rows_v[j, half + t, :]

            acc = lax.fori_loop(0, _L, add_body,
                                jnp.zeros((_D,), jnp.float32))
            x = acc + bvec
            y = lax.rev(x, (0,))
            out_v[c * _CHUNK_SEQ + s, :] = 1.0 / (1.0 + jnp.exp(y - x))
        return carry

    lax.fori_loop(0, _N_CHUNKS, chunk_body, 0)
    pltpu.sync_copy(out_v, out_hbm.at[pl.ds(wid * _SEQ_PER_W, _SEQ_PER_W)])


def kernel(seq, emb, W, b):
    w_pad = jnp.zeros((_EMB, _D), jnp.float32)
    w_pad = w_pad.at[:, 0].set(W[0] * (1.0 / _L))
    w_pad = w_pad.at[:, _D - 1].set(W[1] * (1.0 / _L))
    bvec = jnp.zeros((_D,), jnp.float32)
    bvec = bvec.at[0].set(b[0]).at[_D - 1].set(b[1])
    table = _project_table(emb, w_pad)
    seq3 = seq.astype(jnp.int32).reshape(_NW * _N_CHUNKS, _SUB, _SUB_TOK)
    out16 = _sc_pool(seq3, table, bvec)
    return jnp.stack([out16[:, 0], out16[:, _D - 1]], axis=-1)


# speedup vs baseline: 7.6378x; 1.5295x over previous
import functools

import jax
import jax.numpy as jnp
from jax import lax
from jax.experimental import pallas as pl
from jax.experimental.pallas import tpu as pltpu
from jax.experimental.pallas import tpu_sc as plsc

_VOCAB = 100000
_EMB = 128
_B = 4096
_L = 50
_D = 16
_NC = 2
_NS = 16
_NW = _NC * _NS
_SEQ_PER_W = _B // _NW
_GTOK = 100
_NG = _SEQ_PER_W * _L // _GTOK
_K = 8
_NB = _NG // _K
_BLK_SEQ = _K * _GTOK // _L
_PROJ_BLK = 4000


def _proj_body(emb_ref, w_ref, out_ref):
    out_ref[...] = jnp.dot(emb_ref[...], w_ref[...],
                           preferred_element_type=jnp.float32)


def _project_table(emb, w_pad):
    return pl.pallas_call(
        _proj_body,
        grid=(_VOCAB // _PROJ_BLK,),
        in_specs=[
            pl.BlockSpec((_PROJ_BLK, _EMB), lambda i: (i, 0)),
            pl.BlockSpec((_EMB, _D), lambda i: (0, 0)),
        ],
        out_specs=pl.BlockSpec((_PROJ_BLK, _D), lambda i: (i, 0)),
        out_shape=jax.ShapeDtypeStruct((_VOCAB, _D), jnp.float32),
    )(emb, w_pad)


@functools.partial(
    pl.kernel,
    out_type=jax.ShapeDtypeStruct((_B, _D), jnp.float32),
    mesh=plsc.VectorSubcoreMesh(core_axis_name="c", subcore_axis_name="s",
                                num_cores=_NC, num_subcores=_NS),
    scratch_types=[
        pltpu.VMEM((_NG, _GTOK), jnp.int32),
        pltpu.VMEM((_K, _GTOK, _D), jnp.float32),
        pltpu.VMEM((_K, _GTOK, _D), jnp.float32),
        pltpu.VMEM((_SEQ_PER_W, _D), jnp.float32),
        pltpu.VMEM((_D,), jnp.float32),
        pltpu.SemaphoreType.DMA,
        pltpu.SemaphoreType.DMA,
    ],
    compiler_params=pltpu.CompilerParams(use_tc_tiling_on_sc=False),
)
def _sc_pool(seq_hbm, p_hbm, bvec_hbm, out_hbm, idx_v, rows_a, rows_b,
             out_v, bvec_v, sem_a, sem_b):
    wid = lax.axis_index("s") * _NC + lax.axis_index("c")
    pltpu.sync_copy(bvec_hbm, bvec_v)
    pltpu.sync_copy(seq_hbm.at[pl.ds(wid * _NG, _NG)], idx_v)
    bvec = bvec_v[...]

    def fire(blk, rows, sem):
        for j in range(_K):
            pltpu.async_copy(p_hbm.at[idx_v.at[blk * _K + j]],
                             rows.at[j], sem)

    def drain(blk, rows, sem):
        for j in range(_K):
            pltpu.make_async_copy(p_hbm.at[idx_v.at[blk * _K + j]],
                                  rows.at[j], sem).wait()

    def reduce_block(blk, rows):
        for s in range(_BLK_SEQ):
            g = s // 2
            half = (s % 2) * _L
            accs = [jnp.zeros((_D,), jnp.float32) for _ in range(4)]
            for t in range(_L):
                accs[t % 4] = accs[t % 4] + rows[g, half + t, :]
            x = (accs[0] + accs[1]) + (accs[2] + accs[3]) + bvec
            y = lax.rev(x, (0,))
            out_v[blk * _BLK_SEQ + s, :] = 1.0 / (1.0 + jnp.exp(y - x))

    fire(0, rows_a, sem_a)

    def pipe_body(i, carry):
        b0 = 2 * i
        b1 = 2 * i + 1
        fire(b1, rows_b, sem_b)
        drain(b0, rows_a, sem_a)
        reduce_block(b0, rows_a)

        @pl.when(b1 + 1 < _NB)
        def _():
            fire(b1 + 1, rows_a, sem_a)

        drain(b1, rows_b, sem_b)
        reduce_block(b1, rows_b)
        return carry

    lax.fori_loop(0, _NB // 2, pipe_body, 0)
    pltpu.sync_copy(out_v, out_hbm.at[pl.ds(wid * _SEQ_PER_W, _SEQ_PER_W)])


def kernel(seq, emb, W, b):
    w_pad = jnp.zeros((_EMB, _D), jnp.float32)
    w_pad = w_pad.at[:, 0].set(W[0] * (1.0 / _L))
    w_pad = w_pad.at[:, _D - 1].set(W[1] * (1.0 / _L))
    bvec = jnp.zeros((_D,), jnp.float32)
    bvec = bvec.at[0].set(b[0]).at[_D - 1].set(b[1])
    table = _project_table(emb, w_pad)
    seq2 = seq.astype(jnp.int32).reshape(_NW * _NG, _GTOK)
    out16 = _sc_pool(seq2, table, bvec)
    return jnp.stack([out16[:, 0], out16[:, _D - 1]], axis=-1)


# speedup vs baseline: 11.4883x; 1.5041x over previous
import functools

import jax
import jax.numpy as jnp
from jax import lax
from jax.experimental import pallas as pl
from jax.experimental.pallas import tpu as pltpu
from jax.experimental.pallas import tpu_sc as plsc

_VOCAB = 100000
_EMB = 128
_B = 4096
_L = 50
_NV = _EMB // 16
_NC = 2
_NS = 16
_NW = _NC * _NS
_SEQ_PER_W = _B // _NW
_GTOK = 100
_NG = _SEQ_PER_W * _L // _GTOK
_K = 2
_NB = _NG // _K
_BLK_SEQ = _K * _GTOK // _L
_TAIL_BLK = 512


@functools.partial(
    pl.kernel,
    out_type=jax.ShapeDtypeStruct((_B, _EMB), jnp.float32),
    mesh=plsc.VectorSubcoreMesh(core_axis_name="c", subcore_axis_name="s",
                                num_cores=_NC, num_subcores=_NS),
    scratch_types=[
        pltpu.VMEM((_NG, _GTOK), jnp.int32),
        pltpu.VMEM((_K, _GTOK, _EMB), jnp.float32),
        pltpu.VMEM((_K, _GTOK, _EMB), jnp.float32),
        pltpu.VMEM((_SEQ_PER_W, _EMB), jnp.float32),
        pltpu.SemaphoreType.DMA,
        pltpu.SemaphoreType.DMA,
    ],
    compiler_params=pltpu.CompilerParams(use_tc_tiling_on_sc=False),
)
def _sc_pool(seq_hbm, emb_hbm, out_hbm, idx_v, rows_a, rows_b, out_v,
             sem_a, sem_b):
    wid = lax.axis_index("s") * _NC + lax.axis_index("c")
    pltpu.sync_copy(seq_hbm.at[pl.ds(wid * _NG, _NG)], idx_v)

    def fire(blk, rows, sem):
        for j in range(_K):
            pltpu.async_copy(emb_hbm.at[idx_v.at[blk * _K + j]],
                             rows.at[j], sem)

    def drain(blk, rows, sem):
        for j in range(_K):
            pltpu.make_async_copy(emb_hbm.at[idx_v.at[blk * _K + j]],
                                  rows.at[j], sem).wait()

    def reduce_block(blk, rows):
        for s in range(_BLK_SEQ):
            g = s // 2
            half = (s % 2) * _L

            def add_body(t5, accs):
                new = []
                for v in range(_NV):
                    a = accs[v]
                    for u in range(5):
                        a = a + rows[g, half + t5 * 5 + u,
                                     pl.ds(v * 16, 16)]
                    new.append(a)
                return tuple(new)

            accs = lax.fori_loop(
                0, _L // 5, add_body,
                tuple(jnp.zeros((16,), jnp.float32) for _ in range(_NV)))
            for v in range(_NV):
                out_v[blk * _BLK_SEQ + s, pl.ds(v * 16, 16)] = accs[v]

    fire(0, rows_a, sem_a)

    def pipe_body(i, carry):
        b0 = 2 * i
        b1 = 2 * i + 1
        fire(b1, rows_b, sem_b)
        drain(b0, rows_a, sem_a)
        reduce_block(b0, rows_a)

        @pl.when(b1 + 1 < _NB)
        def _():
            fire(b1 + 1, rows_a, sem_a)

        drain(b1, rows_b, sem_b)
        reduce_block(b1, rows_b)
        return carry

    lax.fori_loop(0, _NB // 2, pipe_body, 0)
    pltpu.sync_copy(out_v, out_hbm.at[pl.ds(wid * _SEQ_PER_W, _SEQ_PER_W)])


def _tail_body(s_ref, w_ref, b_ref, out_ref):
    logits = jnp.dot(s_ref[...], w_ref[...],
                     preferred_element_type=jnp.float32) + b_ref[...]
    out_ref[...] = jax.nn.softmax(logits, axis=-1)


def _dense_tail(sums, wt, b2):
    return pl.pallas_call(
        _tail_body,
        grid=(_B // _TAIL_BLK,),
        in_specs=[
            pl.BlockSpec((_TAIL_BLK, _EMB), lambda i: (i, 0)),
            pl.BlockSpec((_EMB, 2), lambda i: (0, 0)),
            pl.BlockSpec((1, 2), lambda i: (0, 0)),
        ],
        out_specs=pl.BlockSpec((_TAIL_BLK, 2), lambda i: (i, 0)),
        out_shape=jax.ShapeDtypeStruct((_B, 2), jnp.float32),
    )(sums, wt, b2)


def kernel(seq, emb, W, b):
    seq2 = seq.astype(jnp.int32).reshape(_NW * _NG, _GTOK)
    sums = _sc_pool(seq2, emb)
    wt = W.T * (1.0 / _L)
    return _dense_tail(sums, wt, b.reshape(1, 2))


# speedup vs baseline: 11.8875x; 1.0348x over previous
import functools

import jax
import jax.numpy as jnp
from jax import lax
from jax.experimental import pallas as pl
from jax.experimental.pallas import tpu as pltpu
from jax.experimental.pallas import tpu_sc as plsc

_VOCAB = 100000
_EMB = 128
_B = 4096
_L = 50
_NV = _EMB // 16
_NC = 2
_NS = 16
_NW = _NC * _NS
_SEQ_PER_W = _B // _NW
_GTOK = 100
_NG = _SEQ_PER_W * _L // _GTOK
_K = 4
_NB = _NG // _K
_BLK_SEQ = _K * _GTOK // _L
_TAIL_BLK = 512


@functools.partial(
    pl.kernel,
    out_type=jax.ShapeDtypeStruct((_B, _EMB), jnp.float32),
    mesh=plsc.VectorSubcoreMesh(core_axis_name="c", subcore_axis_name="s",
                                num_cores=_NC, num_subcores=_NS),
    scratch_types=[
        pltpu.VMEM((_NG, _GTOK), jnp.int32),
        pltpu.VMEM((_K, _GTOK, _EMB), jnp.float32),
        pltpu.VMEM((_K, _GTOK, _EMB), jnp.float32),
        pltpu.VMEM((_SEQ_PER_W, _EMB), jnp.float32),
        pltpu.SemaphoreType.DMA,
        pltpu.SemaphoreType.DMA,
    ],
    compiler_params=pltpu.CompilerParams(use_tc_tiling_on_sc=False),
)
def _sc_pool(seq_hbm, emb_hbm, out_hbm, idx_v, rows_a, rows_b, out_v,
             sem_a, sem_b):
    wid = lax.axis_index("s") * _NC + lax.axis_index("c")
    pltpu.sync_copy(seq_hbm.at[pl.ds(wid * _NG, _NG)], idx_v)

    def fire(blk, rows, sem):
        for j in range(_K):
            pltpu.async_copy(emb_hbm.at[idx_v.at[blk * _K + j]],
                             rows.at[j], sem)

    def drain(blk, rows, sem):
        for j in range(_K):
            pltpu.make_async_copy(emb_hbm.at[idx_v.at[blk * _K + j]],
                                  rows.at[j], sem).wait()

    def reduce_block(blk, rows):
        for s in range(_BLK_SEQ):
            g = s // 2
            half = (s % 2) * _L

            def add_body(t5, accs):
                new = []
                for v in range(_NV):
                    a = accs[v]
                    for u in range(5):
                        a = a + rows[g, half + t5 * 5 + u,
                                     pl.ds(v * 16, 16)]
                    new.append(a)
                return tuple(new)

            accs = lax.fori_loop(
                0, _L // 5, add_body,
                tuple(jnp.zeros((16,), jnp.float32) for _ in range(_NV)))
            for v in range(_NV):
                out_v[blk * _BLK_SEQ + s, pl.ds(v * 16, 16)] = accs[v]

    fire(0, rows_a, sem_a)

    def pipe_body(i, carry):
        b0 = 2 * i
        b1 = 2 * i + 1
        fire(b1, rows_b, sem_b)
        drain(b0, rows_a, sem_a)
        reduce_block(b0, rows_a)

        @pl.when(b1 + 1 < _NB)
        def _():
            fire(b1 + 1, rows_a, sem_a)

        drain(b1, rows_b, sem_b)
        reduce_block(b1, rows_b)
        return carry

    lax.fori_loop(0, _NB // 2, pipe_body, 0)
    pltpu.sync_copy(out_v, out_hbm.at[pl.ds(wid * _SEQ_PER_W, _SEQ_PER_W)])


def _tail_body(s_ref, w_ref, b_ref, out_ref):
    logits = jnp.dot(s_ref[...], w_ref[...],
                     preferred_element_type=jnp.float32) + b_ref[...]
    out_ref[...] = jax.nn.softmax(logits, axis=-1)


def _dense_tail(sums, wt, b2):
    return pl.pallas_call(
        _tail_body,
        grid=(_B // _TAIL_BLK,),
        in_specs=[
            pl.BlockSpec((_TAIL_BLK, _EMB), lambda i: (i, 0)),
            pl.BlockSpec((_EMB, 2), lambda i: (0, 0)),
            pl.BlockSpec((1, 2), lambda i: (0, 0)),
        ],
        out_specs=pl.BlockSpec((_TAIL_BLK, 2), lambda i: (i, 0)),
        out_shape=jax.ShapeDtypeStruct((_B, 2), jnp.float32),
    )(sums, wt, b2)


def kernel(seq, emb, W, b):
    seq2 = seq.astype(jnp.int32).reshape(_NW * _NG, _GTOK)
    sums = _sc_pool(seq2, emb)
    wt = W.T * (1.0 / _L)
    return _dense_tail(sums, wt, b.reshape(1, 2))
